# baseline (speedup 1.0000x reference)
import functools

import jax
import jax.numpy as jnp
from jax import lax
from jax.experimental import pallas as pl
from jax.experimental.pallas import tpu as pltpu
from jax.experimental.pallas import tpu_sc as plsc

N, C, H, W = 4, 96, 224, 224
HW = H * W

NUM_CORES = 2
NUM_SUBCORES = 16
NWORK = NUM_CORES * NUM_SUBCORES
TILES_PER_BATCH = NWORK // N
C_PER_TILE = C // TILES_PER_BATCH
PAIRS = C_PER_TILE // 2

RPC = 8
CHUNK = RPC * W
NCHUNK = H // RPC
KITER = NCHUNK // 2

_mesh = plsc.VectorSubcoreMesh(
    core_axis_name="c", subcore_axis_name="s",
    num_cores=NUM_CORES, num_subcores=NUM_SUBCORES)


@functools.partial(
    pl.kernel,
    out_type=jax.ShapeDtypeStruct((N * C, H, W), jnp.float32),
    mesh=_mesh,
    compiler_params=pltpu.CompilerParams(needs_layout_passes=False),
    scratch_types=[
        pltpu.VMEM((H, W), jnp.float32),
        pltpu.VMEM((H, W), jnp.float32),
        pltpu.VMEM((CHUNK,), jnp.float32),
        pltpu.VMEM((CHUNK,), jnp.float32),
        pltpu.VMEM((CHUNK,), jnp.float32),
        pltpu.VMEM((CHUNK,), jnp.float32),
        pltpu.VMEM((RPC, W), jnp.float32),
        pltpu.VMEM((RPC, W), jnp.float32),
        pltpu.VMEM((RPC, W), jnp.float32),
        pltpu.VMEM((RPC, W), jnp.float32),
        pltpu.SemaphoreType.DMA,
        pltpu.SemaphoreType.DMA,
        pltpu.SemaphoreType.DMA,
        pltpu.SemaphoreType.DMA,
        pltpu.SemaphoreType.DMA,
    ],
)
def _grid_sample_sc(inp, gx, gy, out, plane_a, plane_b,
                    gxv0, gxv1, gyv0, gyv1, oa0, oa1, ob0, ob1,
                    sem_pl, sem_in0, sem_in1, sem_out0, sem_out1):
    gxv = (gxv0, gxv1)
    gyv = (gyv0, gyv1)
    oa = (oa0, oa1)
    ob = (ob0, ob1)
    wid = lax.axis_index("s") * NUM_CORES + lax.axis_index("c")
    n = wid // TILES_PER_BATCH
    c_base = (wid % TILES_PER_BATCH) * C_PER_TILE
    sem_in = (sem_in0, sem_in1)
    sem_out = (sem_out0, sem_out1)

    def pair_body(p, _):
        row = n * C + c_base + 2 * p
        pa_d = pltpu.async_copy(inp.at[row], plane_a, sem_pl)
        pb_d = pltpu.async_copy(inp.at[row + 1], plane_b, sem_pl)
        for b in (0, 1):
            pltpu.async_copy(gx.at[pl.ds(n * HW + b * CHUNK, CHUNK)],
                             gxv[b], sem_in[b])
            pltpu.async_copy(gy.at[pl.ds(n * HW + b * CHUNK, CHUNK)],
                             gyv[b], sem_in[b])
        pa_d.wait()
        pb_d.wait()

        def kbody(k, _):
            for b in (0, 1):
                i = 2 * k + b
                off = n * HW + i * CHUNK
                r0 = i * RPC
                gxb, gyb = gxv[b], gyv[b]
                oab, obb = oa[b], ob[b]
                pltpu.make_async_copy(
                    gx.at[pl.ds(off, CHUNK)], gxb, sem_in[b]).wait()
                pltpu.make_async_copy(
                    gy.at[pl.ds(off, CHUNK)], gyb, sem_in[b]).wait()

                @pl.when(k >= 1)
                def _():
                    pltpu.make_async_copy(
                        oab, out.at[row, pl.ds(r0, RPC)], sem_out[b]).wait()
                    pltpu.make_async_copy(
                        obb, out.at[row + 1, pl.ds(r0, RPC)],
                        sem_out[b]).wait()

                def row_body(r, _):
                    base = r * W

                    @plsc.parallel_loop(0, W, step=16, unroll=2)
                    def grp(x):
                        s = pl.ds(base + x, 16)
                        ix = (gxb[s] + 1.0) * ((W - 1) * 0.5)
                        iy = (gyb[s] + 1.0) * ((H - 1) * 0.5)
                        ix0 = jnp.maximum(
                            jnp.minimum(ix.astype(jnp.int32), W - 2), 0)
                        iy0 = jnp.maximum(
                            jnp.minimum(iy.astype(jnp.int32), H - 2), 0)
                        fx = ix - ix0.astype(jnp.float32)
                        fy = iy - iy0.astype(jnp.float32)
                        w11 = fx * fy
                        w10 = fy - w11
                        w01 = fx - w11
                        w00 = (1.0 - fx) - w10
                        ix1 = ix0 + 1
                        iy1 = iy0 + 1
                        for plane, ov in ((plane_a, oab), (plane_b, obb)):
                            v00 = plsc.load_gather(plane, [iy0, ix0])
                            v01 = plsc.load_gather(plane, [iy0, ix1])
                            v10 = plsc.load_gather(plane, [iy1, ix0])
                            v11 = plsc.load_gather(plane, [iy1, ix1])
                            ov[r, pl.ds(x, 16)] = (w00 * v00 + w01 * v01
                                                   + w10 * v10 + w11 * v11)

                    return 0

                lax.fori_loop(0, RPC, row_body, 0)
                pltpu.async_copy(oab, out.at[row, pl.ds(r0, RPC)],
                                 sem_out[b])
                pltpu.async_copy(obb, out.at[row + 1, pl.ds(r0, RPC)],
                                 sem_out[b])

                @pl.when(k < KITER - 1)
                def _():
                    off2 = off + 2 * CHUNK
                    pltpu.async_copy(gx.at[pl.ds(off2, CHUNK)],
                                     gxb, sem_in[b])
                    pltpu.async_copy(gy.at[pl.ds(off2, CHUNK)],
                                     gyb, sem_in[b])
            return 0

        lax.fori_loop(0, KITER, kbody, 0)
        for b in (0, 1):
            r0 = (NCHUNK - 2 + b) * RPC
            pltpu.make_async_copy(
                oa[b], out.at[row, pl.ds(r0, RPC)], sem_out[b]).wait()
            pltpu.make_async_copy(
                ob[b], out.at[row + 1, pl.ds(r0, RPC)],
                sem_out[b]).wait()
        return 0

    lax.fori_loop(0, PAIRS, pair_body, 0)


def kernel(input, grid):
    inp = input.reshape(N * C, H, W)
    gx = grid[..., 0].reshape(N * HW)
    gy = grid[..., 1].reshape(N * HW)
    out = _grid_sample_sc(inp, gx, gy)
    return out.reshape(N, C, H, W)


# speedup vs baseline: 1.1293x; 1.1293x over previous
import functools

import jax
import jax.numpy as jnp
from jax import lax
from jax.experimental import pallas as pl
from jax.experimental.pallas import tpu as pltpu
from jax.experimental.pallas import tpu_sc as plsc

N, C, H, W = 4, 96, 224, 224
HW = H * W

NUM_CORES = 2
NUM_SUBCORES = 16
NWORK = NUM_CORES * NUM_SUBCORES
TILES_PER_BATCH = NWORK // N
C_PER_TILE = C // TILES_PER_BATCH
PAIRS = C_PER_TILE // 2

RPC = 8
CHUNK = RPC * W
NCHUNK = H // RPC
KITER = NCHUNK // 2

_mesh = plsc.VectorSubcoreMesh(
    core_axis_name="c", subcore_axis_name="s",
    num_cores=NUM_CORES, num_subcores=NUM_SUBCORES)


@functools.partial(
    pl.kernel,
    out_type=jax.ShapeDtypeStruct((N * C, H, W), jnp.float32),
    mesh=_mesh,
    compiler_params=pltpu.CompilerParams(needs_layout_passes=False),
    scratch_types=[
        pltpu.VMEM((H, W), jnp.float32),
        pltpu.VMEM((H, W), jnp.float32),
        pltpu.VMEM((CHUNK,), jnp.float32),
        pltpu.VMEM((CHUNK,), jnp.float32),
        pltpu.VMEM((CHUNK,), jnp.float32),
        pltpu.VMEM((CHUNK,), jnp.float32),
        pltpu.VMEM((RPC, W), jnp.float32),
        pltpu.VMEM((RPC, W), jnp.float32),
        pltpu.VMEM((RPC, W), jnp.float32),
        pltpu.VMEM((RPC, W), jnp.float32),
        pltpu.SemaphoreType.DMA,
        pltpu.SemaphoreType.DMA,
        pltpu.SemaphoreType.DMA,
        pltpu.SemaphoreType.DMA,
        pltpu.SemaphoreType.DMA,
    ],
)
def _grid_sample_sc(inp, gx, gy, out, plane_a, plane_b,
                    gxv0, gxv1, gyv0, gyv1, oa0, oa1, ob0, ob1,
                    sem_pl, sem_in0, sem_in1, sem_out0, sem_out1):
    gxv = (gxv0, gxv1)
    gyv = (gyv0, gyv1)
    oa = (oa0, oa1)
    ob = (ob0, ob1)
    wid = lax.axis_index("s") * NUM_CORES + lax.axis_index("c")
    n = wid // TILES_PER_BATCH
    c_base = (wid % TILES_PER_BATCH) * C_PER_TILE
    sem_in = (sem_in0, sem_in1)
    sem_out = (sem_out0, sem_out1)

    def pair_body(p, _):
        row = n * C + c_base + 2 * p
        pa_d = pltpu.async_copy(inp.at[row], plane_a, sem_pl)
        pb_d = pltpu.async_copy(inp.at[row + 1], plane_b, sem_pl)
        for b in (0, 1):
            pltpu.async_copy(gx.at[pl.ds(n * HW + b * CHUNK, CHUNK)],
                             gxv[b], sem_in[b])
            pltpu.async_copy(gy.at[pl.ds(n * HW + b * CHUNK, CHUNK)],
                             gyv[b], sem_in[b])
        pa_d.wait()
        pb_d.wait()

        def kbody(k, _):
            for b in (0, 1):
                i = 2 * k + b
                off = n * HW + i * CHUNK
                r0 = i * RPC
                gxb, gyb = gxv[b], gyv[b]
                oab, obb = oa[b], ob[b]
                pltpu.make_async_copy(
                    gx.at[pl.ds(off, CHUNK)], gxb, sem_in[b]).wait()
                pltpu.make_async_copy(
                    gy.at[pl.ds(off, CHUNK)], gyb, sem_in[b]).wait()

                @pl.when(k >= 1)
                def _():
                    pltpu.make_async_copy(
                        oab, out.at[row, pl.ds(r0, RPC)], sem_out[b]).wait()
                    pltpu.make_async_copy(
                        obb, out.at[row + 1, pl.ds(r0, RPC)],
                        sem_out[b]).wait()

                @plsc.parallel_loop(0, RPC * (W // 16), step=1, unroll=2)
                def grp(j):
                    r = jnp.bitwise_and(j, RPC - 1)
                    x = jnp.left_shift(jnp.right_shift(j, 3), 4)
                    if True:
                        s = pl.ds(r * W + x, 16)
                        ix = (gxb[s] + 1.0) * ((W - 1) * 0.5)
                        iy = (gyb[s] + 1.0) * ((H - 1) * 0.5)
                        ix0 = jnp.maximum(
                            jnp.minimum(ix.astype(jnp.int32), W - 2), 0)
                        iy0 = jnp.maximum(
                            jnp.minimum(iy.astype(jnp.int32), H - 2), 0)
                        fx = ix - ix0.astype(jnp.float32)
                        fy = iy - iy0.astype(jnp.float32)
                        w11 = fx * fy
                        w10 = fy - w11
                        w01 = fx - w11
                        w00 = (1.0 - fx) - w10
                        ix1 = ix0 + 1
                        iy1 = iy0 + 1
                        for plane, ov in ((plane_a, oab), (plane_b, obb)):
                            v00 = plsc.load_gather(plane, [iy0, ix0])
                            v01 = plsc.load_gather(plane, [iy0, ix1])
                            v10 = plsc.load_gather(plane, [iy1, ix0])
                            v11 = plsc.load_gather(plane, [iy1, ix1])
                            ov[r, pl.ds(x, 16)] = (w00 * v00 + w01 * v01
                                                   + w10 * v10 + w11 * v11)

                pltpu.async_copy(oab, out.at[row, pl.ds(r0, RPC)],
                                 sem_out[b])
                pltpu.async_copy(obb, out.at[row + 1, pl.ds(r0, RPC)],
                                 sem_out[b])

                @pl.when(k < KITER - 1)
                def _():
                    off2 = off + 2 * CHUNK
                    pltpu.async_copy(gx.at[pl.ds(off2, CHUNK)],
                                     gxb, sem_in[b])
                    pltpu.async_copy(gy.at[pl.ds(off2, CHUNK)],
                                     gyb, sem_in[b])
            return 0

        lax.fori_loop(0, KITER, kbody, 0)
        for b in (0, 1):
            r0 = (NCHUNK - 2 + b) * RPC
            pltpu.make_async_copy(
                oa[b], out.at[row, pl.ds(r0, RPC)], sem_out[b]).wait()
            pltpu.make_async_copy(
                ob[b], out.at[row + 1, pl.ds(r0, RPC)],
                sem_out[b]).wait()
        return 0

    lax.fori_loop(0, PAIRS, pair_body, 0)


def kernel(input, grid):
    inp = input.reshape(N * C, H, W)
    gx = grid[..., 0].reshape(N * HW)
    gy = grid[..., 1].reshape(N * HW)
    out = _grid_sample_sc(inp, gx, gy)
    return out.reshape(N, C, H, W)


# speedup vs baseline: 1.1515x; 1.0197x over previous
import functools

import jax
import jax.numpy as jnp
from jax import lax
from jax.experimental import pallas as pl
from jax.experimental.pallas import tpu as pltpu
from jax.experimental.pallas import tpu_sc as plsc

N, C, H, W = 4, 96, 224, 224
HW = H * W

NUM_CORES = 2
NUM_SUBCORES = 16
NWORK = NUM_CORES * NUM_SUBCORES
TILES_PER_BATCH = NWORK // N
C_PER_TILE = C // TILES_PER_BATCH
PAIRS = C_PER_TILE // 2

RPC = 8
CHUNK = RPC * W
NCHUNK = H // RPC
KITER = NCHUNK // 2

_mesh = plsc.VectorSubcoreMesh(
    core_axis_name="c", subcore_axis_name="s",
    num_cores=NUM_CORES, num_subcores=NUM_SUBCORES)


@functools.partial(
    pl.kernel,
    out_type=jax.ShapeDtypeStruct((N * C, H, W), jnp.float32),
    mesh=_mesh,
    compiler_params=pltpu.CompilerParams(needs_layout_passes=False),
    scratch_types=[
        pltpu.VMEM((H, W), jnp.float32),
        pltpu.VMEM((H, W), jnp.float32),
        pltpu.VMEM((CHUNK,), jnp.float32),
        pltpu.VMEM((CHUNK,), jnp.float32),
        pltpu.VMEM((CHUNK,), jnp.float32),
        pltpu.VMEM((CHUNK,), jnp.float32),
        pltpu.VMEM((RPC, W), jnp.float32),
        pltpu.VMEM((RPC, W), jnp.float32),
        pltpu.VMEM((RPC, W), jnp.float32),
        pltpu.VMEM((RPC, W), jnp.float32),
        pltpu.SemaphoreType.DMA,
        pltpu.SemaphoreType.DMA,
        pltpu.SemaphoreType.DMA,
        pltpu.SemaphoreType.DMA,
        pltpu.SemaphoreType.DMA,
    ],
)
def _grid_sample_sc(inp, gx, gy, out, plane_a, plane_b,
                    gxv0, gxv1, gyv0, gyv1, oa0, oa1, ob0, ob1,
                    sem_pl, sem_in0, sem_in1, sem_out0, sem_out1):
    gxv = (gxv0, gxv1)
    gyv = (gyv0, gyv1)
    oa = (oa0, oa1)
    ob = (ob0, ob1)
    wid = lax.axis_index("s") * NUM_CORES + lax.axis_index("c")
    n = wid // TILES_PER_BATCH
    c_base = (wid % TILES_PER_BATCH) * C_PER_TILE
    sem_in = (sem_in0, sem_in1)
    sem_out = (sem_out0, sem_out1)

    def pair_body(p, _):
        row = n * C + c_base + 2 * p
        pa_d = pltpu.async_copy(inp.at[row], plane_a, sem_pl)
        pb_d = pltpu.async_copy(inp.at[row + 1], plane_b, sem_pl)
        for b in (0, 1):
            pltpu.async_copy(gx.at[pl.ds(n * HW + b * CHUNK, CHUNK)],
                             gxv[b], sem_in[b])
            pltpu.async_copy(gy.at[pl.ds(n * HW + b * CHUNK, CHUNK)],
                             gyv[b], sem_in[b])
        pa_d.wait()
        pb_d.wait()

        def kbody(k, _):
            for b in (0, 1):
                i = 2 * k + b
                off = n * HW + i * CHUNK
                r0 = i * RPC
                gxb, gyb = gxv[b], gyv[b]
                oab, obb = oa[b], ob[b]
                pltpu.make_async_copy(
                    gx.at[pl.ds(off, CHUNK)], gxb, sem_in[b]).wait()
                pltpu.make_async_copy(
                    gy.at[pl.ds(off, CHUNK)], gyb, sem_in[b]).wait()

                @pl.when(k >= 1)
                def _():
                    pltpu.make_async_copy(
                        oab, out.at[row, pl.ds(r0, RPC)], sem_out[b]).wait()
                    pltpu.make_async_copy(
                        obb, out.at[row + 1, pl.ds(r0, RPC)],
                        sem_out[b]).wait()

                @plsc.parallel_loop(0, RPC * (W // 16), step=1, unroll=4)
                def grp(j):
                    r = jnp.bitwise_and(j, RPC - 1)
                    x = jnp.left_shift(jnp.right_shift(j, 3), 4)
                    if True:
                        s = pl.ds(r * W + x, 16)
                        ix = (gxb[s] + 1.0) * ((W - 1) * 0.5)
                        iy = (gyb[s] + 1.0) * ((H - 1) * 0.5)
                        ix0 = jnp.maximum(
                            jnp.minimum(ix.astype(jnp.int32), W - 2), 0)
                        iy0 = jnp.maximum(
                            jnp.minimum(iy.astype(jnp.int32), H - 2), 0)
                        fx = ix - ix0.astype(jnp.float32)
                        fy = iy - iy0.astype(jnp.float32)
                        w11 = fx * fy
                        w10 = fy - w11
                        w01 = fx - w11
                        w00 = (1.0 - fx) - w10
                        ix1 = ix0 + 1
                        iy1 = iy0 + 1
                        for plane, ov in ((plane_a, oab), (plane_b, obb)):
                            v00 = plsc.load_gather(plane, [iy0, ix0])
                            v01 = plsc.load_gather(plane, [iy0, ix1])
                            v10 = plsc.load_gather(plane, [iy1, ix0])
                            v11 = plsc.load_gather(plane, [iy1, ix1])
                            ov[r, pl.ds(x, 16)] = (w00 * v00 + w01 * v01
                                                   + w10 * v10 + w11 * v11)

                pltpu.async_copy(oab, out.at[row, pl.ds(r0, RPC)],
                                 sem_out[b])
                pltpu.async_copy(obb, out.at[row + 1, pl.ds(r0, RPC)],
                                 sem_out[b])

                @pl.when(k < KITER - 1)
                def _():
                    off2 = off + 2 * CHUNK
                    pltpu.async_copy(gx.at[pl.ds(off2, CHUNK)],
                                     gxb, sem_in[b])
                    pltpu.async_copy(gy.at[pl.ds(off2, CHUNK)],
                                     gyb, sem_in[b])
            return 0

        lax.fori_loop(0, KITER, kbody, 0)
        for b in (0, 1):
            r0 = (NCHUNK - 2 + b) * RPC
            pltpu.make_async_copy(
                oa[b], out.at[row, pl.ds(r0, RPC)], sem_out[b]).wait()
            pltpu.make_async_copy(
                ob[b], out.at[row + 1, pl.ds(r0, RPC)],
                sem_out[b]).wait()
        return 0

    lax.fori_loop(0, PAIRS, pair_body, 0)


def kernel(input, grid):
    inp = input.reshape(N * C, H, W)
    gx = grid[..., 0].reshape(N * HW)
    gy = grid[..., 1].reshape(N * HW)
    out = _grid_sample_sc(inp, gx, gy)
    return out.reshape(N, C, H, W)


# speedup vs baseline: 1.1519x; 1.0003x over previous
import functools

import jax
import jax.numpy as jnp
from jax import lax
from jax.experimental import pallas as pl
from jax.experimental.pallas import tpu as pltpu
from jax.experimental.pallas import tpu_sc as plsc

N, C, H, W = 4, 96, 224, 224
HW = H * W

NUM_CORES = 2
NUM_SUBCORES = 16
NWORK = NUM_CORES * NUM_SUBCORES
TILES_PER_BATCH = NWORK // N
C_PER_TILE = C // TILES_PER_BATCH
PAIRS = C_PER_TILE // 2

RPC = 8
CHUNK = RPC * W
NCHUNK = H // RPC
KITER = NCHUNK // 2

_mesh = plsc.VectorSubcoreMesh(
    core_axis_name="c", subcore_axis_name="s",
    num_cores=NUM_CORES, num_subcores=NUM_SUBCORES)


@functools.partial(
    pl.kernel,
    out_type=jax.ShapeDtypeStruct((N * C, H, W), jnp.float32),
    mesh=_mesh,
    compiler_params=pltpu.CompilerParams(needs_layout_passes=False),
    scratch_types=[
        pltpu.VMEM((H, W), jnp.float32),
        pltpu.VMEM((H, W), jnp.float32),
        pltpu.VMEM((CHUNK,), jnp.float32),
        pltpu.VMEM((CHUNK,), jnp.float32),
        pltpu.VMEM((CHUNK,), jnp.float32),
        pltpu.VMEM((CHUNK,), jnp.float32),
        pltpu.VMEM((RPC, W), jnp.float32),
        pltpu.VMEM((RPC, W), jnp.float32),
        pltpu.VMEM((RPC, W), jnp.float32),
        pltpu.VMEM((RPC, W), jnp.float32),
        pltpu.SemaphoreType.DMA,
        pltpu.SemaphoreType.DMA,
        pltpu.SemaphoreType.DMA,
        pltpu.SemaphoreType.DMA,
        pltpu.SemaphoreType.DMA,
    ],
)
def _grid_sample_sc(inp, gx, gy, out, plane_a, plane_b,
                    gxv0, gxv1, gyv0, gyv1, oa0, oa1, ob0, ob1,
                    sem_pl, sem_in0, sem_in1, sem_out0, sem_out1):
    gxv = (gxv0, gxv1)
    gyv = (gyv0, gyv1)
    oa = (oa0, oa1)
    ob = (ob0, ob1)
    wid = lax.axis_index("s") * NUM_CORES + lax.axis_index("c")
    n = wid // TILES_PER_BATCH
    c_base = (wid % TILES_PER_BATCH) * C_PER_TILE
    sem_in = (sem_in0, sem_in1)
    sem_out = (sem_out0, sem_out1)

    def pair_body(p, _):
        row = n * C + c_base + 2 * p
        pa_d = pltpu.async_copy(inp.at[row], plane_a, sem_pl)
        pb_d = pltpu.async_copy(inp.at[row + 1], plane_b, sem_pl)
        for b in (0, 1):
            pltpu.async_copy(gx.at[pl.ds(n * HW + b * CHUNK, CHUNK)],
                             gxv[b], sem_in[b])
            pltpu.async_copy(gy.at[pl.ds(n * HW + b * CHUNK, CHUNK)],
                             gyv[b], sem_in[b])
        pa_d.wait()
        pb_d.wait()

        def kbody(k, _):
            for b in (0, 1):
                i = 2 * k + b
                off = n * HW + i * CHUNK
                r0 = i * RPC
                gxb, gyb = gxv[b], gyv[b]
                oab, obb = oa[b], ob[b]
                pltpu.make_async_copy(
                    gx.at[pl.ds(off, CHUNK)], gxb, sem_in[b]).wait()
                pltpu.make_async_copy(
                    gy.at[pl.ds(off, CHUNK)], gyb, sem_in[b]).wait()

                @pl.when(k >= 1)
                def _():
                    pltpu.make_async_copy(
                        oab, out.at[row, pl.ds(r0, RPC)], sem_out[b]).wait()
                    pltpu.make_async_copy(
                        obb, out.at[row + 1, pl.ds(r0, RPC)],
                        sem_out[b]).wait()

                @plsc.parallel_loop(0, RPC * (W // 16), step=1, unroll=4)
                def grp(j):
                    r = jnp.bitwise_and(j, RPC - 1)
                    x = jnp.left_shift(jnp.right_shift(j, 3), 4)
                    s = pl.ds(r * W + x, 16)
                    ix = (gxb[s] + 1.0) * ((W - 1) * 0.5)
                    iy = (gyb[s] + 1.0) * ((H - 1) * 0.5)
                    ix0 = jnp.maximum(
                        jnp.minimum(ix.astype(jnp.int32), W - 2), 0)
                    iy0 = jnp.maximum(
                        jnp.minimum(iy.astype(jnp.int32), H - 2), 0)
                    fx = ix - ix0.astype(jnp.float32)
                    fy = iy - iy0.astype(jnp.float32)
                    w11 = fx * fy
                    w10 = fy - w11
                    w01 = fx - w11
                    w00 = (1.0 - fx) - w10
                    ix1 = ix0 + 1
                    iy1 = iy0 + 1
                    for plane, ov in ((plane_a, oab), (plane_b, obb)):
                        v00 = plsc.load_gather(plane, [iy0, ix0])
                        v01 = plsc.load_gather(plane, [iy0, ix1])
                        v10 = plsc.load_gather(plane, [iy1, ix0])
                        v11 = plsc.load_gather(plane, [iy1, ix1])
                        ov[r, pl.ds(x, 16)] = (w00 * v00 + w01 * v01
                                               + w10 * v10 + w11 * v11)

                pltpu.async_copy(oab, out.at[row, pl.ds(r0, RPC)],
                                 sem_out[b])
                pltpu.async_copy(obb, out.at[row + 1, pl.ds(r0, RPC)],
                                 sem_out[b])

                @pl.when(k < KITER - 1)
                def _():
                    off2 = off + 2 * CHUNK
                    pltpu.async_copy(gx.at[pl.ds(off2, CHUNK)],
                                     gxb, sem_in[b])
                    pltpu.async_copy(gy.at[pl.ds(off2, CHUNK)],
                                     gyb, sem_in[b])
            return 0

        lax.fori_loop(0, KITER, kbody, 0)
        for b in (0, 1):
            r0 = (NCHUNK - 2 + b) * RPC
            pltpu.make_async_copy(
                oa[b], out.at[row, pl.ds(r0, RPC)], sem_out[b]).wait()
            pltpu.make_async_copy(
                ob[b], out.at[row + 1, pl.ds(r0, RPC)],
                sem_out[b]).wait()
        return 0

    lax.fori_loop(0, PAIRS, pair_body, 0)


def kernel(input, grid):
    inp = input.reshape(N * C, H, W)
    gx = grid[..., 0].reshape(N * HW)
    gy = grid[..., 1].reshape(N * HW)
    out = _grid_sample_sc(inp, gx, gy)
    return out.reshape(N, C, H, W)
